# baseline (speedup 1.0000x reference)
import jax
import jax.numpy as jnp
from jax.experimental import pallas as pl
from jax.experimental.pallas import tpu as pltpu

_DIM = 96
_E = 8
_HID = _DIM
_EPS = 1e-06
_H = 224
_W = 224
_KS = 7
_PAD = 3
_BR = 8


def _fused_block(xa_ref, xb_halo_ref, in_ref, kw_ref, cb_ref, g_ref, b_ref, gw_ref,
                 w1_ref, b1_ref, w2_ref, b2_ref, ls_ref, out_ref):
    rows = jnp.concatenate([xa_ref[...], xb_halo_ref[...]], axis=0)
    rows = rows.astype(jnp.bfloat16)
    kw = kw_ref[...]

    gw = gw_ref[...]
    w1 = w1_ref[...]
    w2 = w2_ref[...]
    b1 = b1_ref[...]
    b2t = b2_ref[...]
    g = g_ref[...]
    b = b_ref[...]

    win = {}
    for j in range(_BR + _KS - 1):
        slab = jax.lax.slice(rows, (j, 0, 0), (j + 1, _DIM, _W + _KS - 1))
        slab = slab.reshape(_DIM, _W + _KS - 1)
        for dw in range(_KS):
            win[(j, dw)] = jax.lax.slice(slab, (0, dw), (_DIM, dw + _W))

    ys = []
    for r in range(_BR):
        acc = jnp.zeros((_DIM, _W), dtype=jnp.bfloat16)
        for dh in range(_KS):
            for dw in range(_KS):
                k = jax.lax.slice(kw, (0, dh * _KS + dw), (_DIM, dh * _KS + dw + 1))
                acc = acc + win[(r + dh, dw)] * k
        acc = acc.astype(jnp.float32) + cb_ref[...]
        mu = jnp.mean(acc, axis=0, keepdims=True)
        var = jnp.mean(acc * acc, axis=0, keepdims=True) - mu * mu
        xln = (acc - mu) * jax.lax.rsqrt(var + _EPS) * g + b
        xb = xln.astype(jnp.bfloat16)
        logits = jnp.dot(gw, xb, preferred_element_type=jnp.float32)
        mx = jnp.max(logits, axis=0, keepdims=True)
        iota = jax.lax.broadcasted_iota(jnp.int32, (_E, _W), 0)
        eidx = jnp.min(jnp.where(logits == mx, iota, _E), axis=0, keepdims=True)
        hall = jnp.dot(w1, xb, preferred_element_type=jnp.float32)
        hsel = jnp.zeros((_HID, _W), dtype=jnp.float32)
        b2sel = jnp.zeros((_DIM, _W), dtype=jnp.float32)
        for e in range(_E):
            m = (eidx == e).astype(jnp.float32)
            h_e = jax.lax.slice(hall, (e * _HID, 0), ((e + 1) * _HID, _W))
            b1_e = jax.lax.slice(b1, (e * _HID, 0), ((e + 1) * _HID, 1))
            hsel = hsel + m * (h_e + b1_e)
            b2sel = b2sel + m * jax.lax.slice(b2t, (0, e), (_DIM, e + 1))
        hact = jax.nn.gelu(hsel, approximate=True).astype(jnp.bfloat16)
        pieces = []
        for e in range(_E):
            m16 = (eidx == e).astype(jnp.bfloat16)
            pieces.append(m16 * hact)
        h2 = jnp.concatenate(pieces, axis=0)
        y_r = jnp.dot(w2, h2, preferred_element_type=jnp.float32) + b2sel
        ys.append(y_r[None, :, :])
    y3 = jnp.concatenate(ys, axis=0)

    out_ref[...] = in_ref[...] + ls_ref[...][None, :, :] * y3


def kernel(input, conv_w, conv_b, ln_g, ln_b, gate_w, w1, b1, w2, b2, ls):
    x = jnp.transpose(input[0], (1, 0, 2))
    xpa = jnp.pad(x[: _H - _PAD], ((_PAD, 0), (0, 0), (_PAD, _PAD)))
    xpb = jnp.pad(x[_BR - _PAD:], ((0, _BR - _PAD), (0, 0), (_PAD, _PAD)))
    kw = conv_w.reshape(_DIM, _KS * _KS).astype(jnp.bfloat16)
    cb2 = conv_b[:, None]
    g2 = ln_g[:, None]
    b2d = ln_b[:, None]
    gwb = gate_w.astype(jnp.bfloat16)
    w1b = w1.reshape(_E * _HID, _DIM).astype(jnp.bfloat16)
    w2b = jnp.transpose(w2, (1, 0, 2)).reshape(_DIM, _E * _HID).astype(jnp.bfloat16)
    b1c = b1.reshape(_E * _HID, 1)
    b2t = b2.T
    ls2 = ls[:, 0, :]

    grid = (_H // _BR,)
    out = pl.pallas_call(
        _fused_block,
        grid=grid,
        in_specs=[
            pl.BlockSpec((_BR, _DIM, _W + 2 * _PAD), lambda i: (i, 0, 0)),
            pl.BlockSpec((_BR, _DIM, _W + 2 * _PAD), lambda i: (i, 0, 0)),
            pl.BlockSpec((_BR, _DIM, _W), lambda i: (i, 0, 0)),
            pl.BlockSpec(kw.shape, lambda i: (0, 0)),
            pl.BlockSpec(cb2.shape, lambda i: (0, 0)),
            pl.BlockSpec(g2.shape, lambda i: (0, 0)),
            pl.BlockSpec(b2d.shape, lambda i: (0, 0)),
            pl.BlockSpec(gwb.shape, lambda i: (0, 0)),
            pl.BlockSpec(w1b.shape, lambda i: (0, 0)),
            pl.BlockSpec(b1c.shape, lambda i: (0, 0)),
            pl.BlockSpec(w2b.shape, lambda i: (0, 0)),
            pl.BlockSpec(b2t.shape, lambda i: (0, 0)),
            pl.BlockSpec(ls2.shape, lambda i: (0, 0)),
        ],
        out_specs=pl.BlockSpec((_BR, _DIM, _W), lambda i: (i, 0, 0)),
        out_shape=jax.ShapeDtypeStruct((_H, _DIM, _W), jnp.float32),
        compiler_params=pltpu.CompilerParams(
            dimension_semantics=("arbitrary",),
            vmem_limit_bytes=100 * 1024 * 1024,
        ),
    )(xpa, xpb, x, kw, cb2, g2, b2d, gwb, w1b, b1c, w2b, b2t, ls2)

    return jnp.transpose(out, (1, 0, 2))[None]


# speedup vs baseline: 1.3295x; 1.3295x over previous
import jax
import jax.numpy as jnp
from jax.experimental import pallas as pl
from jax.experimental.pallas import tpu as pltpu

_DIM = 96
_E = 8
_HID = _DIM
_EPS = 1e-06
_H = 224
_W = 224
_KS = 7
_PAD = 3
_BR = 16


def _fused_block(xpad_ref, in_ref, kw_ref, cb_ref, g_ref, b_ref, gw_ref,
                 w1_ref, b1_ref, w2_ref, b2_ref, ls_ref, out_ref):
    i = pl.program_id(0)
    rows = xpad_ref[pl.ds(i * _BR, _BR + _KS - 1), :, :]
    rows = rows.astype(jnp.bfloat16)
    kw = kw_ref[...]

    gw = gw_ref[...]
    w1 = w1_ref[...]
    w2 = w2_ref[...]
    b1 = b1_ref[...]
    b2t = b2_ref[...]
    g = g_ref[...]
    b = b_ref[...]

    win = {}
    for j in range(_BR + _KS - 1):
        slab = jax.lax.slice(rows, (j, 0, 0), (j + 1, _DIM, _W + _KS - 1))
        slab = slab.reshape(_DIM, _W + _KS - 1)
        for dw in range(_KS):
            win[(j, dw)] = jax.lax.slice(slab, (0, dw), (_DIM, dw + _W))

    ys = []
    for r in range(_BR):
        acc = jnp.zeros((_DIM, _W), dtype=jnp.bfloat16)
        for dh in range(_KS):
            for dw in range(_KS):
                k = jax.lax.slice(kw, (0, dh * _KS + dw), (_DIM, dh * _KS + dw + 1))
                acc = acc + win[(r + dh, dw)] * k
        acc = acc.astype(jnp.float32) + cb_ref[...]
        mu = jnp.mean(acc, axis=0, keepdims=True)
        var = jnp.mean(acc * acc, axis=0, keepdims=True) - mu * mu
        xln = (acc - mu) * jax.lax.rsqrt(var + _EPS) * g + b
        xb = xln.astype(jnp.bfloat16)
        logits = jnp.dot(gw, xb, preferred_element_type=jnp.float32)
        mx = jnp.max(logits, axis=0, keepdims=True)
        iota = jax.lax.broadcasted_iota(jnp.int32, (_E, _W), 0)
        eidx = jnp.min(jnp.where(logits == mx, iota, _E), axis=0, keepdims=True)
        hall = jnp.dot(w1, xb, preferred_element_type=jnp.float32)
        hsel = jnp.zeros((_HID, _W), dtype=jnp.float32)
        b2sel = jnp.zeros((_DIM, _W), dtype=jnp.float32)
        for e in range(_E):
            m = (eidx == e).astype(jnp.float32)
            h_e = jax.lax.slice(hall, (e * _HID, 0), ((e + 1) * _HID, _W))
            b1_e = jax.lax.slice(b1, (e * _HID, 0), ((e + 1) * _HID, 1))
            hsel = hsel + m * (h_e + b1_e)
            b2sel = b2sel + m * jax.lax.slice(b2t, (0, e), (_DIM, e + 1))
        hact = jax.nn.gelu(hsel, approximate=True).astype(jnp.bfloat16)
        pieces = []
        for e in range(_E):
            m16 = (eidx == e).astype(jnp.bfloat16)
            pieces.append(m16 * hact)
        h2 = jnp.concatenate(pieces, axis=0)
        y_r = jnp.dot(w2, h2, preferred_element_type=jnp.float32) + b2sel
        ys.append(y_r[None, :, :])
    y3 = jnp.concatenate(ys, axis=0)

    out_ref[...] = in_ref[...] + ls_ref[...][None, :, :] * y3


def kernel(input, conv_w, conv_b, ln_g, ln_b, gate_w, w1, b1, w2, b2, ls):
    x = jnp.transpose(input[0], (1, 0, 2))
    xpad = jnp.pad(x, ((_PAD, _PAD), (0, 0), (_PAD, _PAD)))
    kw = conv_w.reshape(_DIM, _KS * _KS).astype(jnp.bfloat16)
    cb2 = conv_b[:, None]
    g2 = ln_g[:, None]
    b2d = ln_b[:, None]
    gwb = gate_w.astype(jnp.bfloat16)
    w1b = w1.reshape(_E * _HID, _DIM).astype(jnp.bfloat16)
    w2b = jnp.transpose(w2, (1, 0, 2)).reshape(_DIM, _E * _HID).astype(jnp.bfloat16)
    b1c = b1.reshape(_E * _HID, 1)
    b2t = b2.T
    ls2 = ls[:, 0, :]

    grid = (_H // _BR,)
    out = pl.pallas_call(
        _fused_block,
        grid=grid,
        in_specs=[
            pl.BlockSpec(xpad.shape, lambda i: (0, 0, 0)),
            pl.BlockSpec((_BR, _DIM, _W), lambda i: (i, 0, 0)),
            pl.BlockSpec(kw.shape, lambda i: (0, 0)),
            pl.BlockSpec(cb2.shape, lambda i: (0, 0)),
            pl.BlockSpec(g2.shape, lambda i: (0, 0)),
            pl.BlockSpec(b2d.shape, lambda i: (0, 0)),
            pl.BlockSpec(gwb.shape, lambda i: (0, 0)),
            pl.BlockSpec(w1b.shape, lambda i: (0, 0)),
            pl.BlockSpec(b1c.shape, lambda i: (0, 0)),
            pl.BlockSpec(w2b.shape, lambda i: (0, 0)),
            pl.BlockSpec(b2t.shape, lambda i: (0, 0)),
            pl.BlockSpec(ls2.shape, lambda i: (0, 0)),
        ],
        out_specs=pl.BlockSpec((_BR, _DIM, _W), lambda i: (i, 0, 0)),
        out_shape=jax.ShapeDtypeStruct((_H, _DIM, _W), jnp.float32),
        compiler_params=pltpu.CompilerParams(
            dimension_semantics=("arbitrary",),
            vmem_limit_bytes=100 * 1024 * 1024,
        ),
    )(xpad, x, kw, cb2, g2, b2d, gwb, w1b, b1c, w2b, b2t, ls2)

    return jnp.transpose(out, (1, 0, 2))[None]
